# baseline (speedup 1.0000x reference)
import jax
import jax.numpy as jnp
from jax import lax
from jax.experimental import pallas as pl
from jax.experimental.pallas import tpu as pltpu
from jax.experimental.pallas import tpu_sc as plsc

_R = 3
_N = 10000
_NP = 10240
_E = 320000
_EP = 327680
_EROWS = _EP // 128
_TROWS = _EROWS // 16
_MR = 16
_MACROS = _TROWS // _MR
_ACC = _R * _NP
_STRIPE = _ACC // 16
_QW = 32


_NBUF = 4
_SLACK = 1
_GR = 32
_GROUPS = _TROWS // _GR


def _sc_body(xf, cols, offs, sums, hists,
             offb, gixb, rows, hist,
             isems, jsems, gsems, ssems, acc):
    c = lax.axis_index("c")
    s = lax.axis_index("s")
    zeros16 = jnp.zeros((16,), jnp.float32)
    ones16 = jnp.ones((16,), jnp.float32)

    def zh(r, carry):
        hist[pl.ds(r * 16, 16)] = zeros16
        return carry
    lax.fori_loop(0, 2048, zh, 0)

    base_a = s * _STRIPE

    for p in range(2):
        q = p * 2 + c

        def z0(r, carry):
            for w in range(2):
                rows[r, pl.ds(w * 16, 16)] = zeros16
            return carry
        lax.fori_loop(0, 128, z0, 0)

        zd = [pltpu.async_copy(rows.at[pl.ds(0, 128)],
                               acc.at[pl.ds(base_a + k * 128, 128)],
                               ssems.at[k % _NBUF])
              for k in range(_STRIPE // 128)]
        for d in zd:
            d.wait()

        plsc.subcore_barrier()

        qn = q * _N
        tbase = s * _TROWS

        pltpu.async_copy(offs.at[pl.ds(tbase, _GR)], offb.at[0],
                         isems.at[0])
        pltpu.async_copy(cols.at[pl.ds(tbase, _GR)], gixb.at[0],
                         jsems.at[0])

        def group(m, carry):
            par = lax.rem(m, 2)
            pltpu.make_async_copy(offs.at[pl.ds(tbase, _GR)],
                                  offb.at[par], isems.at[par]).wait()
            pltpu.make_async_copy(cols.at[pl.ds(tbase, _GR)],
                                  gixb.at[par], jsems.at[par]).wait()

            @pl.when(m + 1 < _GROUPS)
            def _():
                nbase = tbase + (m + 1) * _GR
                pltpu.async_copy(offs.at[pl.ds(nbase, _GR)],
                                 offb.at[1 - par], isems.at[1 - par])
                pltpu.async_copy(cols.at[pl.ds(nbase, _GR)],
                                 gixb.at[1 - par], jsems.at[1 - par])

            def addrow(r):
                for w in range(8):
                    gixb[par, r, pl.ds(w * 16, 16)] = (
                        gixb[par, r, pl.ds(w * 16, 16)] + qn)

            gd, sd = {}, {}
            for b in range(_NBUF):
                addrow(b)
                gd[b] = pltpu.async_copy(
                    xf.at[gixb.at[par, b]],
                    rows.at[pl.ds(b * 128, 128)], gsems.at[b])
            for j in range(_GR):
                b = j % _NBUF
                gd[j].wait()
                sd[j] = pltpu.async_copy(
                    rows.at[pl.ds(b * 128, 128)],
                    acc.at[offb.at[par, j]], ssems.at[b], add=True)
                if p == 0:
                    for w in range(8):
                        o = offb[par, j, pl.ds(w * 16, 16)]
                        plsc.addupdate_scatter(hist, [o], ones16)
                jj = j - _SLACK
                nxt = jj + _NBUF
                if jj >= 0 and nxt < _GR:
                    sd[jj].wait()
                    addrow(nxt)
                    gd[nxt] = pltpu.async_copy(
                        xf.at[gixb.at[par, nxt]],
                        rows.at[pl.ds((jj % _NBUF) * 128, 128)],
                        gsems.at[jj % _NBUF])
            for j in range(_GR - _NBUF, _GR):
                sd[j].wait()
            return carry
        lax.fori_loop(0, _GROUPS, group, 0)

        plsc.subcore_barrier()

        pltpu.sync_copy(acc.at[pl.ds(base_a, _STRIPE)],
                        sums.at[pl.ds(base_a, _STRIPE),
                                pl.ds(q * _QW, _QW)])

    @pl.when(c == 0)
    def _():
        pltpu.sync_copy(hist, hists.at[s])


_sc_call = pl.kernel(
    _sc_body,
    out_type=[
        jax.ShapeDtypeStruct((_ACC, 128), jnp.float32),
        jax.ShapeDtypeStruct((16, 32768), jnp.float32),
    ],
    mesh=plsc.VectorSubcoreMesh(core_axis_name="c", subcore_axis_name="s"),
    compiler_params=pltpu.CompilerParams(
        needs_layout_passes=False, use_tc_tiling_on_sc=False),
    scratch_types=[
        pltpu.VMEM((2, _GR, 128), jnp.int32),
        pltpu.VMEM((2, _GR, 128), jnp.int32),
        pltpu.VMEM((_NBUF * 128, _QW), jnp.float32),
        pltpu.VMEM((32768,), jnp.float32),
        pltpu.SemaphoreType.DMA((2,)),
        pltpu.SemaphoreType.DMA((2,)),
        pltpu.SemaphoreType.DMA((_NBUF,)),
        pltpu.SemaphoreType.DMA((_NBUF,)),
        pltpu.VMEM_SHARED((_ACC, _QW), jnp.float32),
    ],
)


def _fin_body(x_ref, s_ref, cnt_ref, o_ref):
    cnt48 = cnt_ref[...]
    x = x_ref[...]
    cnt = [cnt48[:, r:r + 1] for r in range(_R)]
    for t in range(1, 16):
        for r in range(_R):
            cnt[r] = cnt[r] + cnt48[:, 3 * t + r:3 * t + r + 1]
    c0 = jnp.maximum(cnt[0], 1.0)
    c1 = jnp.maximum(cnt[1], 1.0)
    c2 = jnp.maximum(cnt[2], 1.0)
    ctx = s_ref[0] / c0 + s_ref[1] / c1 + s_ref[2] / c2
    rc = (jnp.minimum(cnt[0], 1.0) + jnp.minimum(cnt[1], 1.0)
          + jnp.minimum(cnt[2], 1.0))
    rc = jnp.maximum(rc, 1.0)
    o_ref[...] = x - ctx / rc


_FB = 1000

_fin_call = pl.pallas_call(
    _fin_body,
    grid=(_N // _FB,),
    in_specs=[
        pl.BlockSpec((_FB, 128), lambda i: (i, 0)),
        pl.BlockSpec((_R, _FB, 128), lambda i: (0, i, 0)),
        pl.BlockSpec((_FB, 48), lambda i: (i, 0)),
    ],
    out_specs=pl.BlockSpec((_FB, 128), lambda i: (i, 0)),
    out_shape=jax.ShapeDtypeStruct((_N, 128), jnp.float32),
)


def kernel(x, edge_index, edge_type):
    row = edge_index[0]
    col = edge_index[1]
    off = edge_type * _NP + row
    pad = _EP - _E
    dummy_off = _N + (jnp.arange(pad, dtype=jnp.int32) % 16)
    offp = jnp.concatenate([off, dummy_off]).reshape(_EROWS, 128)
    colp = jnp.concatenate(
        [col, jnp.zeros((pad,), jnp.int32)]).reshape(_EROWS, 128)
    xf = x.reshape(_N, 4, _QW).transpose(1, 0, 2).reshape(4 * _N, _QW)

    sums, hists = _sc_call(xf, colp, offp)
    s = sums.reshape(_R, _NP, 128)
    cnt = (hists[:, :_ACC].reshape(16, _R, _NP)
           .transpose(2, 0, 1).reshape(_NP, 48))
    out = _fin_call(x, s, cnt)
    return out


# speedup vs baseline: 1.0116x; 1.0116x over previous
import jax
import jax.numpy as jnp
from jax import lax
from jax.experimental import pallas as pl
from jax.experimental.pallas import tpu as pltpu
from jax.experimental.pallas import tpu_sc as plsc

_R = 3
_N = 10000
_NP = 10240
_E = 320000
_EP = 327680
_EROWS = _EP // 128
_TROWS = _EROWS // 16
_MR = 16
_MACROS = _TROWS // _MR
_ACC = _R * _NP
_STRIPE = _ACC // 16
_QW = 32


_NBUF = 5
_SLACK = 2
_GR = 32
_GROUPS = _TROWS // _GR


def _sc_body(xf, cols, offs, sums, hists,
             offb, gixb, rows, hist,
             isems, jsems, gsems, ssems, acc):
    c = lax.axis_index("c")
    s = lax.axis_index("s")
    zeros16 = jnp.zeros((16,), jnp.float32)
    ones16 = jnp.ones((16,), jnp.float32)

    def zh(r, carry):
        hist[pl.ds(r * 16, 16)] = zeros16
        return carry
    lax.fori_loop(0, _ACC // 16, zh, 0)

    base_a = s * _STRIPE

    for p in range(2):
        q = p * 2 + c

        def z0(r, carry):
            for w in range(2):
                rows[r, pl.ds(w * 16, 16)] = zeros16
            return carry
        lax.fori_loop(0, 128, z0, 0)

        zd = [pltpu.async_copy(rows.at[pl.ds(0, 128)],
                               acc.at[pl.ds(base_a + k * 128, 128)],
                               ssems.at[k % _NBUF])
              for k in range(_STRIPE // 128)]
        for d in zd:
            d.wait()

        plsc.subcore_barrier()

        qn = q * _N
        tbase = s * _TROWS

        pltpu.async_copy(offs.at[pl.ds(tbase, _GR)], offb.at[0],
                         isems.at[0])
        pltpu.async_copy(cols.at[pl.ds(tbase, _GR)], gixb.at[0],
                         jsems.at[0])

        def group(m, carry):
            par = lax.rem(m, 2)
            pltpu.make_async_copy(offs.at[pl.ds(tbase, _GR)],
                                  offb.at[par], isems.at[par]).wait()
            pltpu.make_async_copy(cols.at[pl.ds(tbase, _GR)],
                                  gixb.at[par], jsems.at[par]).wait()

            @pl.when(m + 1 < _GROUPS)
            def _():
                nbase = tbase + (m + 1) * _GR
                pltpu.async_copy(offs.at[pl.ds(nbase, _GR)],
                                 offb.at[1 - par], isems.at[1 - par])
                pltpu.async_copy(cols.at[pl.ds(nbase, _GR)],
                                 gixb.at[1 - par], jsems.at[1 - par])

            def addrow(r):
                for w in range(8):
                    gixb[par, r, pl.ds(w * 16, 16)] = (
                        gixb[par, r, pl.ds(w * 16, 16)] + qn)

            gd, sd = {}, {}
            for b in range(_NBUF):
                addrow(b)
                gd[b] = pltpu.async_copy(
                    xf.at[gixb.at[par, b]],
                    rows.at[pl.ds(b * 128, 128)], gsems.at[b])
            for j in range(_GR):
                b = j % _NBUF
                gd[j].wait()
                sd[j] = pltpu.async_copy(
                    rows.at[pl.ds(b * 128, 128)],
                    acc.at[offb.at[par, j]], ssems.at[b], add=True)
                if p == 0:
                    for w in range(8):
                        o = offb[par, j, pl.ds(w * 16, 16)]
                        plsc.addupdate_scatter(hist, [o], ones16)
                jj = j - _SLACK
                nxt = jj + _NBUF
                if jj >= 0 and nxt < _GR:
                    sd[jj].wait()
                    addrow(nxt)
                    gd[nxt] = pltpu.async_copy(
                        xf.at[gixb.at[par, nxt]],
                        rows.at[pl.ds((jj % _NBUF) * 128, 128)],
                        gsems.at[jj % _NBUF])
            for j in range(_GR - _NBUF, _GR):
                sd[j].wait()
            return carry
        lax.fori_loop(0, _GROUPS, group, 0)

        plsc.subcore_barrier()

        pltpu.sync_copy(acc.at[pl.ds(base_a, _STRIPE)],
                        sums.at[pl.ds(base_a, _STRIPE),
                                pl.ds(q * _QW, _QW)])

    @pl.when(c == 0)
    def _():
        pltpu.sync_copy(hist, hists.at[s])


_sc_call = pl.kernel(
    _sc_body,
    out_type=[
        jax.ShapeDtypeStruct((_ACC, 128), jnp.float32),
        jax.ShapeDtypeStruct((16, _ACC), jnp.float32),
    ],
    mesh=plsc.VectorSubcoreMesh(core_axis_name="c", subcore_axis_name="s"),
    compiler_params=pltpu.CompilerParams(
        needs_layout_passes=False, use_tc_tiling_on_sc=False),
    scratch_types=[
        pltpu.VMEM((2, _GR, 128), jnp.int32),
        pltpu.VMEM((2, _GR, 128), jnp.int32),
        pltpu.VMEM((_NBUF * 128, _QW), jnp.float32),
        pltpu.VMEM((_ACC,), jnp.float32),
        pltpu.SemaphoreType.DMA((2,)),
        pltpu.SemaphoreType.DMA((2,)),
        pltpu.SemaphoreType.DMA((_NBUF,)),
        pltpu.SemaphoreType.DMA((_NBUF,)),
        pltpu.VMEM_SHARED((_ACC, _QW), jnp.float32),
    ],
)


def _fin_body(x_ref, s_ref, cnt_ref, o_ref):
    cnt48 = cnt_ref[...]
    x = x_ref[...]
    cnt = [cnt48[:, r:r + 1] for r in range(_R)]
    for t in range(1, 16):
        for r in range(_R):
            cnt[r] = cnt[r] + cnt48[:, 3 * t + r:3 * t + r + 1]
    c0 = jnp.maximum(cnt[0], 1.0)
    c1 = jnp.maximum(cnt[1], 1.0)
    c2 = jnp.maximum(cnt[2], 1.0)
    ctx = s_ref[0] / c0 + s_ref[1] / c1 + s_ref[2] / c2
    rc = (jnp.minimum(cnt[0], 1.0) + jnp.minimum(cnt[1], 1.0)
          + jnp.minimum(cnt[2], 1.0))
    rc = jnp.maximum(rc, 1.0)
    o_ref[...] = x - ctx / rc


_FB = 1000

_fin_call = pl.pallas_call(
    _fin_body,
    grid=(_N // _FB,),
    in_specs=[
        pl.BlockSpec((_FB, 128), lambda i: (i, 0)),
        pl.BlockSpec((_R, _FB, 128), lambda i: (0, i, 0)),
        pl.BlockSpec((_FB, 48), lambda i: (i, 0)),
    ],
    out_specs=pl.BlockSpec((_FB, 128), lambda i: (i, 0)),
    out_shape=jax.ShapeDtypeStruct((_N, 128), jnp.float32),
)


def kernel(x, edge_index, edge_type):
    row = edge_index[0]
    col = edge_index[1]
    off = edge_type * _NP + row
    pad = _EP - _E
    dummy_off = _N + (jnp.arange(pad, dtype=jnp.int32) % 16)
    offp = jnp.concatenate([off, dummy_off]).reshape(_EROWS, 128)
    colp = jnp.concatenate(
        [col, jnp.zeros((pad,), jnp.int32)]).reshape(_EROWS, 128)
    xf = x.reshape(_N, 4, _QW).transpose(1, 0, 2).reshape(4 * _N, _QW)

    sums, hists = _sc_call(xf, colp, offp)
    s = sums.reshape(_R, _NP, 128)
    cnt = (hists.reshape(16, _R, _NP)
           .transpose(2, 0, 1).reshape(_NP, 48))
    out = _fin_call(x, s, cnt)
    return out


# speedup vs baseline: 1.3877x; 1.3718x over previous
import jax
import jax.numpy as jnp
from jax import lax
from jax.experimental import pallas as pl
from jax.experimental.pallas import tpu as pltpu
from jax.experimental.pallas import tpu_sc as plsc

_R = 3
_N = 10000
_NP = 10240
_E = 320000
_EP = 327680
_EROWS = _EP // 128
_TROWS = _EROWS // 16
_MR = 16
_MACROS = _TROWS // _MR
_ACC = _R * _NP
_STRIPE = _ACC // 16
_HW = 64


_NBUF = 5
_SLACK = 2
_GR = 32
_GROUPS = _TROWS // _GR


def _sc_body(xf, cols, offs, sums, hists,
             offb, gixb, rows, hist,
             isems, jsems, gsems, ssems, acc):
    c = lax.axis_index("c")
    s = lax.axis_index("s")
    zeros16 = jnp.zeros((16,), jnp.float32)
    zeros32b = jnp.zeros((32,), jnp.bfloat16)
    ones16 = jnp.ones((16,), jnp.float32)

    def zh(r, carry):
        hist[pl.ds(r * 16, 16)] = zeros16
        return carry
    lax.fori_loop(0, _ACC // 16, zh, 0)

    base_a = s * _STRIPE

    for p in range(1):
        def z0(r, carry):
            for w in range(2):
                rows[r, pl.ds(w * 32, 32)] = zeros32b
            return carry
        lax.fori_loop(0, 128, z0, 0)

        zd = [pltpu.async_copy(rows.at[pl.ds(0, 128)],
                               acc.at[pl.ds(base_a + k * 128, 128)],
                               ssems.at[k % _NBUF])
              for k in range(_STRIPE // 128)]
        for d in zd:
            d.wait()

        plsc.subcore_barrier()

        qn = c * _N
        tbase = s * _TROWS

        pltpu.async_copy(offs.at[pl.ds(tbase, _GR)], offb.at[0],
                         isems.at[0])
        pltpu.async_copy(cols.at[pl.ds(tbase, _GR)], gixb.at[0],
                         jsems.at[0])

        def group(m, carry):
            par = lax.rem(m, 2)
            pltpu.make_async_copy(offs.at[pl.ds(tbase, _GR)],
                                  offb.at[par], isems.at[par]).wait()
            pltpu.make_async_copy(cols.at[pl.ds(tbase, _GR)],
                                  gixb.at[par], jsems.at[par]).wait()

            @pl.when(m + 1 < _GROUPS)
            def _():
                nbase = tbase + (m + 1) * _GR
                pltpu.async_copy(offs.at[pl.ds(nbase, _GR)],
                                 offb.at[1 - par], isems.at[1 - par])
                pltpu.async_copy(cols.at[pl.ds(nbase, _GR)],
                                 gixb.at[1 - par], jsems.at[1 - par])

            def addrow(r):
                for w in range(8):
                    gixb[par, r, pl.ds(w * 16, 16)] = (
                        gixb[par, r, pl.ds(w * 16, 16)] + qn)

            gd, sd = {}, {}
            for b in range(_NBUF):
                addrow(b)
                gd[b] = pltpu.async_copy(
                    xf.at[gixb.at[par, b]],
                    rows.at[pl.ds(b * 128, 128)], gsems.at[b])
            for j in range(_GR):
                b = j % _NBUF
                gd[j].wait()
                sd[j] = pltpu.async_copy(
                    rows.at[pl.ds(b * 128, 128)],
                    acc.at[offb.at[par, j]], ssems.at[b], add=True)
                if p == 0:
                    for w in range(8):
                        o = offb[par, j, pl.ds(w * 16, 16)]
                        plsc.addupdate_scatter(hist, [o], ones16)
                jj = j - _SLACK
                nxt = jj + _NBUF
                if jj >= 0 and nxt < _GR:
                    sd[jj].wait()
                    addrow(nxt)
                    gd[nxt] = pltpu.async_copy(
                        xf.at[gixb.at[par, nxt]],
                        rows.at[pl.ds((jj % _NBUF) * 128, 128)],
                        gsems.at[jj % _NBUF])
            for j in range(_GR - _NBUF, _GR):
                sd[j].wait()
            return carry
        lax.fori_loop(0, _GROUPS, group, 0)

        plsc.subcore_barrier()

        pltpu.sync_copy(acc.at[pl.ds(base_a, _STRIPE)],
                        sums.at[pl.ds(base_a, _STRIPE),
                                pl.ds(c * _HW, _HW)])

    @pl.when(c == 0)
    def _():
        pltpu.sync_copy(hist, hists.at[s])


_sc_call = pl.kernel(
    _sc_body,
    out_type=[
        jax.ShapeDtypeStruct((_ACC, 128), jnp.bfloat16),
        jax.ShapeDtypeStruct((16, _ACC), jnp.float32),
    ],
    mesh=plsc.VectorSubcoreMesh(core_axis_name="c", subcore_axis_name="s"),
    compiler_params=pltpu.CompilerParams(
        needs_layout_passes=False, use_tc_tiling_on_sc=False),
    scratch_types=[
        pltpu.VMEM((2, _GR, 128), jnp.int32),
        pltpu.VMEM((2, _GR, 128), jnp.int32),
        pltpu.VMEM((_NBUF * 128, _HW), jnp.bfloat16),
        pltpu.VMEM((_ACC,), jnp.float32),
        pltpu.SemaphoreType.DMA((2,)),
        pltpu.SemaphoreType.DMA((2,)),
        pltpu.SemaphoreType.DMA((_NBUF,)),
        pltpu.SemaphoreType.DMA((_NBUF,)),
        pltpu.VMEM_SHARED((_ACC, _HW), jnp.bfloat16),
    ],
)


def _fin_body(x_ref, s_ref, cnt_ref, o_ref):
    cnt48 = cnt_ref[...]
    x = x_ref[...]
    cnt = [cnt48[:, r:r + 1] for r in range(_R)]
    for t in range(1, 16):
        for r in range(_R):
            cnt[r] = cnt[r] + cnt48[:, 3 * t + r:3 * t + r + 1]
    c0 = jnp.maximum(cnt[0], 1.0)
    c1 = jnp.maximum(cnt[1], 1.0)
    c2 = jnp.maximum(cnt[2], 1.0)
    ctx = s_ref[0] / c0 + s_ref[1] / c1 + s_ref[2] / c2
    rc = (jnp.minimum(cnt[0], 1.0) + jnp.minimum(cnt[1], 1.0)
          + jnp.minimum(cnt[2], 1.0))
    rc = jnp.maximum(rc, 1.0)
    o_ref[...] = x - ctx / rc


_FB = 1000

_fin_call = pl.pallas_call(
    _fin_body,
    grid=(_N // _FB,),
    in_specs=[
        pl.BlockSpec((_FB, 128), lambda i: (i, 0)),
        pl.BlockSpec((_R, _FB, 128), lambda i: (0, i, 0)),
        pl.BlockSpec((_FB, 48), lambda i: (i, 0)),
    ],
    out_specs=pl.BlockSpec((_FB, 128), lambda i: (i, 0)),
    out_shape=jax.ShapeDtypeStruct((_N, 128), jnp.float32),
)


def kernel(x, edge_index, edge_type):
    row = edge_index[0]
    col = edge_index[1]
    off = edge_type * _NP + row
    pad = _EP - _E
    dummy_off = _N + (jnp.arange(pad, dtype=jnp.int32) % 16)
    offp = jnp.concatenate([off, dummy_off]).reshape(_EROWS, 128)
    colp = jnp.concatenate(
        [col, jnp.zeros((pad,), jnp.int32)]).reshape(_EROWS, 128)
    xf = (x.reshape(_N, 2, _HW).transpose(1, 0, 2)
          .reshape(2 * _N, _HW).astype(jnp.bfloat16))

    sums, hists = _sc_call(xf, colp, offp)
    s = sums.astype(jnp.float32).reshape(_R, _NP, 128)
    cnt = (hists.reshape(16, _R, _NP)
           .transpose(2, 0, 1).reshape(_NP, 48))
    out = _fin_call(x, s, cnt)
    return out


# speedup vs baseline: 1.5910x; 1.1465x over previous
import jax
import jax.numpy as jnp
from jax import lax
from jax.experimental import pallas as pl
from jax.experimental.pallas import tpu as pltpu
from jax.experimental.pallas import tpu_sc as plsc

_R = 3
_N = 10000
_NP = 10240
_E = 320000
_EP = 327680
_EROWS = _EP // 128
_TROWS = _EROWS // 16
_MR = 16
_MACROS = _TROWS // _MR
_ACC = _R * _NP
_STRIPE = _ACC // 16
_HW = 64


_NBUF = 5
_SLACK = 2
_GR = 32
_GROUPS = _TROWS // _GR


def _sc_body(xf, cols, offs, sums, hists,
             offb, gixb, rows, hist,
             isems, jsems, gsems, ssems, acc):
    c = lax.axis_index("c")
    s = lax.axis_index("s")
    zeros16 = jnp.zeros((16,), jnp.float32)
    zeros32b = jnp.zeros((32,), jnp.bfloat16)
    ones16 = jnp.ones((16,), jnp.float32)

    def zh(r, carry):
        hist[pl.ds(r * 16, 16)] = zeros16
        return carry
    lax.fori_loop(0, _ACC // 16, zh, 0)

    base_a = s * _STRIPE

    for p in range(1):
        def z0(r, carry):
            for w in range(2):
                rows[r, pl.ds(w * 32, 32)] = zeros32b
            return carry
        lax.fori_loop(0, 128, z0, 0)

        zd = [pltpu.async_copy(rows.at[pl.ds(0, 128)],
                               acc.at[pl.ds(base_a + k * 128, 128)],
                               ssems.at[k % _NBUF])
              for k in range(_STRIPE // 128)]
        for d in zd:
            d.wait()

        plsc.subcore_barrier()

        qn = c * _N
        tbase = s * _TROWS

        pltpu.async_copy(offs.at[pl.ds(tbase, _GR)], offb.at[0],
                         isems.at[0])
        pltpu.async_copy(cols.at[pl.ds(tbase, _GR)], gixb.at[0],
                         jsems.at[0])

        def group(m, carry):
            par = lax.rem(m, 2)
            pltpu.make_async_copy(offs.at[pl.ds(tbase, _GR)],
                                  offb.at[par], isems.at[par]).wait()
            pltpu.make_async_copy(cols.at[pl.ds(tbase, _GR)],
                                  gixb.at[par], jsems.at[par]).wait()

            @pl.when(m + 1 < _GROUPS)
            def _():
                nbase = tbase + (m + 1) * _GR
                pltpu.async_copy(offs.at[pl.ds(nbase, _GR)],
                                 offb.at[1 - par], isems.at[1 - par])
                pltpu.async_copy(cols.at[pl.ds(nbase, _GR)],
                                 gixb.at[1 - par], jsems.at[1 - par])

            def addrow(r):
                for w in range(8):
                    gixb[par, r, pl.ds(w * 16, 16)] = (
                        gixb[par, r, pl.ds(w * 16, 16)] + qn)

            gd, sd = {}, {}
            for b in range(_NBUF):
                addrow(b)
                gd[b] = pltpu.async_copy(
                    xf.at[gixb.at[par, b]],
                    rows.at[pl.ds(b * 128, 128)], gsems.at[b])
            for j in range(_GR):
                b = j % _NBUF
                gd[j].wait()
                sd[j] = pltpu.async_copy(
                    rows.at[pl.ds(b * 128, 128)],
                    acc.at[offb.at[par, j]], ssems.at[b], add=True)
                if p == 0:
                    for w in range(8):
                        o = offb[par, j, pl.ds(w * 16, 16)]
                        plsc.addupdate_scatter(hist, [o], ones16)
                jj = j - _SLACK
                nxt = jj + _NBUF
                if jj >= 0 and nxt < _GR:
                    sd[jj].wait()
                    addrow(nxt)
                    gd[nxt] = pltpu.async_copy(
                        xf.at[gixb.at[par, nxt]],
                        rows.at[pl.ds((jj % _NBUF) * 128, 128)],
                        gsems.at[jj % _NBUF])
            for j in range(_GR - _NBUF, _GR):
                sd[j].wait()
            return carry
        lax.fori_loop(0, _GROUPS, group, 0)

        plsc.subcore_barrier()

        pltpu.sync_copy(acc.at[pl.ds(base_a, _STRIPE)],
                        sums.at[pl.ds(base_a, _STRIPE),
                                pl.ds(c * _HW, _HW)])

    @pl.when(c == 0)
    def _():
        pltpu.sync_copy(hist, hists.at[s])


_sc_call = pl.kernel(
    _sc_body,
    out_type=[
        jax.ShapeDtypeStruct((_ACC, 128), jnp.bfloat16),
        jax.ShapeDtypeStruct((16, _ACC), jnp.float32),
    ],
    mesh=plsc.VectorSubcoreMesh(core_axis_name="c", subcore_axis_name="s"),
    compiler_params=pltpu.CompilerParams(
        needs_layout_passes=False, use_tc_tiling_on_sc=False),
    scratch_types=[
        pltpu.VMEM((2, _GR, 128), jnp.int32),
        pltpu.VMEM((2, _GR, 128), jnp.int32),
        pltpu.VMEM((_NBUF * 128, _HW), jnp.bfloat16),
        pltpu.VMEM((_ACC,), jnp.float32),
        pltpu.SemaphoreType.DMA((2,)),
        pltpu.SemaphoreType.DMA((2,)),
        pltpu.SemaphoreType.DMA((_NBUF,)),
        pltpu.SemaphoreType.DMA((_NBUF,)),
        pltpu.VMEM_SHARED((_ACC, _HW), jnp.bfloat16),
    ],
)


def _fin_body(x_ref, s_ref, h0_ref, h1_ref, h2_ref, o_ref):
    x = x_ref[...]
    cnt = [jnp.sum(h_ref[...], axis=0)[:, None]
           for h_ref in (h0_ref, h1_ref, h2_ref)]
    c0 = jnp.maximum(cnt[0], 1.0)
    c1 = jnp.maximum(cnt[1], 1.0)
    c2 = jnp.maximum(cnt[2], 1.0)
    ctx = (s_ref[0][...].astype(jnp.float32) / c0
           + s_ref[1][...].astype(jnp.float32) / c1
           + s_ref[2][...].astype(jnp.float32) / c2)
    rc = (jnp.minimum(cnt[0], 1.0) + jnp.minimum(cnt[1], 1.0)
          + jnp.minimum(cnt[2], 1.0))
    rc = jnp.maximum(rc, 1.0)
    o_ref[...] = x - ctx / rc


_FB = 1024

_fin_call = pl.pallas_call(
    _fin_body,
    grid=(_NP // _FB,),
    in_specs=[
        pl.BlockSpec((_FB, 128), lambda i: (i, 0)),
        pl.BlockSpec((_R, _FB, 128), lambda i: (0, i, 0)),
        pl.BlockSpec((16, _FB), lambda i: (0, i)),
        pl.BlockSpec((16, _FB), lambda i: (0, 10 + i)),
        pl.BlockSpec((16, _FB), lambda i: (0, 20 + i)),
    ],
    out_specs=pl.BlockSpec((_FB, 128), lambda i: (i, 0)),
    out_shape=jax.ShapeDtypeStruct((_N, 128), jnp.float32),
)


def kernel(x, edge_index, edge_type):
    row = edge_index[0]
    col = edge_index[1]
    off = edge_type * _NP + row
    pad = _EP - _E
    dummy_off = _N + (jnp.arange(pad, dtype=jnp.int32) % 16)
    offp = jnp.concatenate([off, dummy_off]).reshape(_EROWS, 128)
    colp = jnp.concatenate(
        [col, jnp.zeros((pad,), jnp.int32)]).reshape(_EROWS, 128)
    xf = (x.reshape(_N, 2, _HW).transpose(1, 0, 2)
          .reshape(2 * _N, _HW).astype(jnp.bfloat16))

    sums, hists = _sc_call(xf, colp, offp)
    s = sums.reshape(_R, _NP, 128)
    out = _fin_call(x, s, hists, hists, hists)
    return out
